# speedup vs baseline: 1.1976x; 1.1976x over previous
import jax
import jax.numpy as jnp
from jax.experimental import pallas as pl
from jax.experimental.pallas import tpu as pltpu

PH = 16
PW = 16
HALF = 512


def _body(row_ref, col_ref, out_ref):
    col = col_ref[:]
    for r in range(PH):
        out_ref[r * PW:(r + 1) * PW, :HALF] = jnp.broadcast_to(
            row_ref[r:r + 1, :], (PW, HALF))
        out_ref[r * PW:(r + 1) * PW, HALF:] = col


def kernel(row_embed, col_embed):
    return pl.pallas_call(
        _body,
        out_shape=jax.ShapeDtypeStruct((PH * PW, 2 * HALF), jnp.float32),
        in_specs=[
            pl.BlockSpec(memory_space=pltpu.VMEM),
            pl.BlockSpec(memory_space=pltpu.VMEM),
        ],
        out_specs=pl.BlockSpec(memory_space=pltpu.VMEM),
    )(row_embed, col_embed)
